# Initial kernel scaffold; baseline (speedup 1.0000x reference)
import functools

import jax
import jax.numpy as jnp
from jax import lax
from jax.experimental import pallas as pl
from jax.experimental.pallas import tpu as pltpu
from jax.experimental.pallas import tpu_sc as plsc

N = 10000
E = 160000
D = 256
HALF = 128
NC = 2
NS = 16
EPT = E // NS
CH = 128
NCH = (EPT + CH - 1) // CH
EPAD = NCH * CH
TRASH = N
ACC_ROWS = N + 16
RPT = N // NS


def _sc_agg_body(xh, esrc, edst, out, idx_buf, dst_buf, rows, acc, sem):
    c = lax.axis_index("c")
    s = lax.axis_index("s")
    w = c * NS + s
    base = s * RPT
    pltpu.sync_copy(xh.at[pl.ds(c * N + base, RPT)], acc.at[pl.ds(base, RPT)])
    pltpu.sync_copy(esrc.at[w], idx_buf)
    pltpu.sync_copy(edst.at[w], dst_buf)
    plsc.subcore_barrier()

    def body(j, carry):
        pltpu.async_copy(xh.at[idx_buf.at[j]], rows, sem).wait()
        pltpu.sync_copy(rows, acc.at[dst_buf.at[j]], add=True)
        return carry

    lax.fori_loop(0, NCH, body, 0)
    plsc.subcore_barrier()
    pltpu.sync_copy(acc.at[pl.ds(base, RPT)], out.at[pl.ds(c * N + base, RPT)])


_sc_aggregate = functools.partial(
    pl.kernel,
    out_type=jax.ShapeDtypeStruct((NC * N, HALF), jnp.float32),
    mesh=plsc.VectorSubcoreMesh(
        core_axis_name="c", subcore_axis_name="s", num_cores=NC, num_subcores=NS
    ),
    scratch_types=[
        pltpu.VMEM((NCH, CH), jnp.int32),
        pltpu.VMEM((NCH, CH), jnp.int32),
        pltpu.VMEM((CH, HALF), jnp.float32),
        pltpu.VMEM_SHARED((ACC_ROWS, HALF), jnp.float32),
        pltpu.SemaphoreType.DMA,
    ],
)(_sc_agg_body)


BN = 1000


def _mlp_body_split(a_ref, w1_ref, b1_ref, w2_ref, b2_ref, o_ref):
    h = jnp.dot(a_ref[0], w1_ref[0], preferred_element_type=jnp.float32)
    h = h + jnp.dot(a_ref[1], w1_ref[1], preferred_element_type=jnp.float32)
    h = jnp.maximum(h + b1_ref[...], 0.0)
    o_ref[0] = jnp.dot(h, w2_ref[:, :HALF], preferred_element_type=jnp.float32) + b2_ref[:, :HALF]
    o_ref[1] = jnp.dot(h, w2_ref[:, HALF:], preferred_element_type=jnp.float32) + b2_ref[:, HALF:]


def _mlp_body_final(a_ref, w1_ref, b1_ref, w2_ref, b2_ref, o_ref):
    h = jnp.dot(a_ref[0], w1_ref[0], preferred_element_type=jnp.float32)
    h = h + jnp.dot(a_ref[1], w1_ref[1], preferred_element_type=jnp.float32)
    h = jnp.maximum(h + b1_ref[...], 0.0)
    o_ref[...] = jnp.dot(h, w2_ref[...], preferred_element_type=jnp.float32) + b2_ref[...]


def _mlp_call(split, aggh, W1, b1, W2, b2):
    a3 = aggh.reshape(NC, N, HALF)
    w1 = W1.reshape(NC, HALF, D)
    b1r = b1.reshape(1, D)
    b2r = b2.reshape(1, D)
    in_specs = [
        pl.BlockSpec((NC, BN, HALF), lambda i: (0, i, 0)),
        pl.BlockSpec((NC, HALF, D), lambda i: (0, 0, 0)),
        pl.BlockSpec((1, D), lambda i: (0, 0)),
        pl.BlockSpec((D, D), lambda i: (0, 0)),
        pl.BlockSpec((1, D), lambda i: (0, 0)),
    ]
    if split:
        out_shape = jax.ShapeDtypeStruct((NC, N, HALF), jnp.float32)
        out_spec = pl.BlockSpec((NC, BN, HALF), lambda i: (0, i, 0))
        body = _mlp_body_split
    else:
        out_shape = jax.ShapeDtypeStruct((N, D), jnp.float32)
        out_spec = pl.BlockSpec((BN, D), lambda i: (i, 0))
        body = _mlp_body_final
    return pl.pallas_call(
        body,
        grid=(N // BN,),
        in_specs=in_specs,
        out_specs=out_spec,
        out_shape=out_shape,
    )(a3, w1, b1r, W2, b2r)


def kernel(x, edge_index, W1_0, b1_0, W2_0, b2_0, W1_1, b1_1, W2_1, b2_1,
           W1_2, b1_2, W2_2, b2_2):
    src = edge_index[0]
    dst = edge_index[1]
    src2 = jnp.concatenate([src, src + N]).reshape(NC, NS, EPT)
    src2 = jnp.pad(src2, ((0, 0), (0, 0), (0, EPAD - EPT)))
    esrc = src2.reshape(NC * NS, NCH, CH)
    d3 = jnp.broadcast_to(dst.reshape(1, NS, EPT), (NC, NS, EPT))
    d3 = jnp.pad(d3, ((0, 0), (0, 0), (0, EPAD - EPT)), constant_values=TRASH)
    edst = d3.reshape(NC * NS, NCH, CH)

    xh = x.reshape(N, NC, HALF).transpose(1, 0, 2).reshape(NC * N, HALF)

    params = [(W1_0, b1_0, W2_0, b2_0), (W1_1, b1_1, W2_1, b2_1),
              (W1_2, b1_2, W2_2, b2_2)]
    for l, (W1, b1, W2, b2) in enumerate(params):
        aggh = _sc_aggregate(xh, esrc, edst)
        if l < 2:
            xh = _mlp_call(True, aggh, W1, b1, W2, b2).reshape(NC * N, HALF)
        else:
            return _mlp_call(False, aggh, W1, b1, W2, b2)


# speedup vs baseline: 3.5393x; 3.5393x over previous
import functools

import jax
import jax.numpy as jnp
from jax import lax
from jax.experimental import pallas as pl
from jax.experimental.pallas import tpu as pltpu
from jax.experimental.pallas import tpu_sc as plsc

N = 10000
E = 160000
D = 256
HALF = 128
NC = 2
NS = 16
EPT = E // NS
CH = 128
NCH = (EPT + CH - 1) // CH
EPAD = NCH * CH
TRASH = N
ACC_ROWS = N + 16
RPT = 624
REM = N - NS * RPT


def _sc_agg_body(xh, esrc, edst, out, idx_buf, dst_buf, rows, acc, sem):
    c = lax.axis_index("c")
    s = lax.axis_index("s")
    w = c * NS + s
    base = s * RPT
    pltpu.sync_copy(xh.at[pl.ds(c * N + base, RPT)], acc.at[pl.ds(base, RPT)])

    @pl.when(s == 0)
    def _init_tail():
        pltpu.sync_copy(xh.at[pl.ds(c * N + NS * RPT, REM)],
                        acc.at[pl.ds(NS * RPT, REM)])
    pltpu.sync_copy(esrc.at[w], idx_buf)
    pltpu.sync_copy(edst.at[w], dst_buf)
    plsc.subcore_barrier()

    def body(j, carry):
        pltpu.async_copy(xh.at[idx_buf.at[j]], rows, sem).wait()
        pltpu.sync_copy(rows, acc.at[dst_buf.at[j]], add=True)
        return carry

    lax.fori_loop(0, NCH, body, 0)
    plsc.subcore_barrier()
    pltpu.sync_copy(acc.at[pl.ds(base, RPT)], out.at[pl.ds(c * N + base, RPT)])

    @pl.when(s == 0)
    def _write_tail():
        pltpu.sync_copy(acc.at[pl.ds(NS * RPT, REM)],
                        out.at[pl.ds(c * N + NS * RPT, REM)])


_sc_aggregate = functools.partial(
    pl.kernel,
    out_type=jax.ShapeDtypeStruct((NC * N, HALF), jnp.float32),
    mesh=plsc.VectorSubcoreMesh(
        core_axis_name="c", subcore_axis_name="s", num_cores=NC, num_subcores=NS
    ),
    scratch_types=[
        pltpu.VMEM((NCH, CH), jnp.int32),
        pltpu.VMEM((NCH, CH), jnp.int32),
        pltpu.VMEM((CH, HALF), jnp.float32),
        pltpu.VMEM_SHARED((ACC_ROWS, HALF), jnp.float32),
        pltpu.SemaphoreType.DMA,
    ],
)(_sc_agg_body)


BN = 1000


def _mlp_body_split(a_ref, w1_ref, b1_ref, w2_ref, b2_ref, o_ref):
    h = jnp.dot(a_ref[0], w1_ref[0], preferred_element_type=jnp.float32)
    h = h + jnp.dot(a_ref[1], w1_ref[1], preferred_element_type=jnp.float32)
    h = jnp.maximum(h + b1_ref[...], 0.0)
    o_ref[0] = jnp.dot(h, w2_ref[:, :HALF], preferred_element_type=jnp.float32) + b2_ref[:, :HALF]
    o_ref[1] = jnp.dot(h, w2_ref[:, HALF:], preferred_element_type=jnp.float32) + b2_ref[:, HALF:]


def _mlp_body_final(a_ref, w1_ref, b1_ref, w2_ref, b2_ref, o_ref):
    h = jnp.dot(a_ref[0], w1_ref[0], preferred_element_type=jnp.float32)
    h = h + jnp.dot(a_ref[1], w1_ref[1], preferred_element_type=jnp.float32)
    h = jnp.maximum(h + b1_ref[...], 0.0)
    o_ref[...] = jnp.dot(h, w2_ref[...], preferred_element_type=jnp.float32) + b2_ref[...]


def _mlp_call(split, aggh, W1, b1, W2, b2):
    a3 = aggh.reshape(NC, N, HALF)
    w1 = W1.reshape(NC, HALF, D)
    b1r = b1.reshape(1, D)
    b2r = b2.reshape(1, D)
    in_specs = [
        pl.BlockSpec((NC, BN, HALF), lambda i: (0, i, 0)),
        pl.BlockSpec((NC, HALF, D), lambda i: (0, 0, 0)),
        pl.BlockSpec((1, D), lambda i: (0, 0)),
        pl.BlockSpec((D, D), lambda i: (0, 0)),
        pl.BlockSpec((1, D), lambda i: (0, 0)),
    ]
    if split:
        out_shape = jax.ShapeDtypeStruct((NC, N, HALF), jnp.float32)
        out_spec = pl.BlockSpec((NC, BN, HALF), lambda i: (0, i, 0))
        body = _mlp_body_split
    else:
        out_shape = jax.ShapeDtypeStruct((N, D), jnp.float32)
        out_spec = pl.BlockSpec((BN, D), lambda i: (i, 0))
        body = _mlp_body_final
    return pl.pallas_call(
        body,
        grid=(N // BN,),
        in_specs=in_specs,
        out_specs=out_spec,
        out_shape=out_shape,
    )(a3, w1, b1r, W2, b2r)


def kernel(x, edge_index, W1_0, b1_0, W2_0, b2_0, W1_1, b1_1, W2_1, b2_1,
           W1_2, b1_2, W2_2, b2_2):
    src = edge_index[0]
    dst = edge_index[1]
    src2 = jnp.concatenate([src, src + N]).reshape(NC, NS, EPT)
    src2 = jnp.pad(src2, ((0, 0), (0, 0), (0, EPAD - EPT)))
    esrc = src2.reshape(NC * NS, NCH, CH)
    d3 = jnp.broadcast_to(dst.reshape(1, NS, EPT), (NC, NS, EPT))
    d3 = jnp.pad(d3, ((0, 0), (0, 0), (0, EPAD - EPT)), constant_values=TRASH)
    edst = d3.reshape(NC * NS, NCH, CH)

    xh = x.reshape(N, NC, HALF).transpose(1, 0, 2).reshape(NC * N, HALF)

    params = [(W1_0, b1_0, W2_0, b2_0), (W1_1, b1_1, W2_1, b2_1),
              (W1_2, b1_2, W2_2, b2_2)]
    for l, (W1, b1, W2, b2) in enumerate(params):
        aggh = _sc_aggregate(xh, esrc, edst)
        if l < 2:
            xh = _mlp_call(True, aggh, W1, b1, W2, b2).reshape(NC * N, HALF)
        else:
            return _mlp_call(False, aggh, W1, b1, W2, b2)
